# baseline (speedup 1.0000x reference)
import jax
import jax.numpy as jnp
from jax import lax
from jax.experimental import pallas as pl
from jax.experimental.pallas import tpu as pltpu
from jax.experimental.pallas import tpu_sc as plsc

NUM_ROWS = 100000
EMB_DIM = 3072
BATCH = 8192
CHANNEL, IM_H, IM_W = 3, 32, 32

NUM_CORES = 2
NUM_SUBCORES = 16
NUM_WORKERS = NUM_CORES * NUM_SUBCORES
NSPLIT = 8
FSLICE = EMB_DIM // NSPLIT
ROWS_PER_WORKER = BATCH // NUM_WORKERS
CHUNK = 64
NCHUNK = ROWS_PER_WORKER // CHUNK

TC_BBLK = 512


def _make_sc_gather(split):
    f0 = split * FSLICE

    def body(idx_hbm, table_hbm, out_hbm, idx_v, buf0, buf1, sem0, sem1):
        wid = lax.axis_index("s") * NUM_CORES + lax.axis_index("c")
        base = wid * ROWS_PER_WORKER
        pltpu.sync_copy(idx_hbm.at[pl.ds(base, ROWS_PER_WORKER)], idx_v)

        bufs = (buf0, buf1)
        sems = (sem0, sem1)

        def start_gather(g):
            return pltpu.async_copy(
                table_hbm.at[idx_v.at[pl.ds(g * CHUNK, CHUNK)],
                             pl.ds(f0, FSLICE)],
                bufs[g % 2], sems[g % 2])

        pending = start_gather(0)
        for g in range(NCHUNK):
            nxt = start_gather(g + 1) if g + 1 < NCHUNK else None
            pending.wait()
            pltpu.sync_copy(bufs[g % 2],
                            out_hbm.at[pl.ds(base + g * CHUNK, CHUNK)])
            pending = nxt

    mesh = plsc.VectorSubcoreMesh(core_axis_name="c", subcore_axis_name="s")
    return pl.kernel(
        body,
        out_type=jax.ShapeDtypeStruct((BATCH, FSLICE), jnp.float32),
        mesh=mesh,
        scratch_types=[
            pltpu.VMEM((ROWS_PER_WORKER,), jnp.int32),
            pltpu.VMEM((CHUNK, FSLICE), jnp.float32),
            pltpu.VMEM((CHUNK, FSLICE), jnp.float32),
            pltpu.SemaphoreType.DMA,
            pltpu.SemaphoreType.DMA,
        ],
    )


def _tc_transpose_first_body(x_ref, o_ref):
    o_ref[...] = x_ref[...].T


def _tc_transpose_band_body(x_ref, acc_ref, o_ref):
    del acc_ref
    o_ref[...] = x_ref[...].T


def _make_tc_transpose(split):
    if split == 0:
        return pl.pallas_call(
            _tc_transpose_first_body,
            grid=(BATCH // TC_BBLK,),
            in_specs=[pl.BlockSpec((TC_BBLK, FSLICE), lambda i: (i, 0))],
            out_specs=pl.BlockSpec((FSLICE, TC_BBLK), lambda i: (0, i)),
            out_shape=jax.ShapeDtypeStruct((EMB_DIM, BATCH), jnp.float32),
        )
    return pl.pallas_call(
        _tc_transpose_band_body,
        grid=(BATCH // TC_BBLK,),
        in_specs=[
            pl.BlockSpec((TC_BBLK, FSLICE), lambda i: (i, 0)),
            pl.BlockSpec(memory_space=pltpu.MemorySpace.HBM),
        ],
        out_specs=pl.BlockSpec((FSLICE, TC_BBLK), lambda i, s=split: (s, i)),
        out_shape=jax.ShapeDtypeStruct((EMB_DIM, BATCH), jnp.float32),
        input_output_aliases={1: 0},
    )


def kernel(indices, table):
    idx = indices.astype(jnp.int32)
    parts = [_make_sc_gather(k)(idx, table) for k in range(NSPLIT)]
    out_t = _make_tc_transpose(0)(parts[0])
    for k in range(1, NSPLIT):
        out_t = _make_tc_transpose(k)(parts[k], out_t)
    return out_t.T.reshape(BATCH, CHANNEL, IM_H, IM_W)


# speedup vs baseline: 1.2068x; 1.2068x over previous
import jax
import jax.numpy as jnp
from jax import lax
from jax.experimental import pallas as pl
from jax.experimental.pallas import tpu as pltpu
from jax.experimental.pallas import tpu_sc as plsc

NUM_ROWS = 100000
EMB_DIM = 3072
BATCH = 8192
CHANNEL, IM_H, IM_W = 3, 32, 32

NUM_CORES = 2
NUM_SUBCORES = 16
NUM_WORKERS = NUM_CORES * NUM_SUBCORES
NSPLIT = 3
FSLICE = EMB_DIM // NSPLIT
ROWS_PER_WORKER = BATCH // NUM_WORKERS
CHUNK = 32768 // FSLICE
NCHUNK = ROWS_PER_WORKER // CHUNK

TC_BBLK = 512


def _make_sc_gather(split):
    f0 = split * FSLICE

    def body(idx_hbm, table_hbm, out_hbm, idx_v, buf0, buf1, sem0, sem1):
        wid = lax.axis_index("s") * NUM_CORES + lax.axis_index("c")
        base = wid * ROWS_PER_WORKER
        pltpu.sync_copy(idx_hbm.at[pl.ds(base, ROWS_PER_WORKER)], idx_v)

        bufs = (buf0, buf1)
        sems = (sem0, sem1)

        def start_gather(g):
            return pltpu.async_copy(
                table_hbm.at[idx_v.at[pl.ds(g * CHUNK, CHUNK)],
                             pl.ds(f0, FSLICE)],
                bufs[g % 2], sems[g % 2])

        pending = start_gather(0)
        for g in range(NCHUNK):
            nxt = start_gather(g + 1) if g + 1 < NCHUNK else None
            pending.wait()
            pltpu.sync_copy(bufs[g % 2],
                            out_hbm.at[pl.ds(base + g * CHUNK, CHUNK)])
            pending = nxt

    mesh = plsc.VectorSubcoreMesh(core_axis_name="c", subcore_axis_name="s")
    return pl.kernel(
        body,
        out_type=jax.ShapeDtypeStruct((BATCH, FSLICE), jnp.float32),
        mesh=mesh,
        scratch_types=[
            pltpu.VMEM((ROWS_PER_WORKER,), jnp.int32),
            pltpu.VMEM((CHUNK, FSLICE), jnp.float32),
            pltpu.VMEM((CHUNK, FSLICE), jnp.float32),
            pltpu.SemaphoreType.DMA,
            pltpu.SemaphoreType.DMA,
        ],
    )


def _tc_transpose_first_body(x_ref, o_ref):
    o_ref[...] = x_ref[...].T


def _tc_transpose_band_body(x_ref, acc_ref, o_ref):
    del acc_ref
    o_ref[...] = x_ref[...].T


def _make_tc_transpose(split):
    if split == 0:
        return pl.pallas_call(
            _tc_transpose_first_body,
            grid=(BATCH // TC_BBLK,),
            in_specs=[pl.BlockSpec((TC_BBLK, FSLICE), lambda i: (i, 0))],
            out_specs=pl.BlockSpec((FSLICE, TC_BBLK), lambda i: (0, i)),
            out_shape=jax.ShapeDtypeStruct((EMB_DIM, BATCH), jnp.float32),
        )
    return pl.pallas_call(
        _tc_transpose_band_body,
        grid=(BATCH // TC_BBLK,),
        in_specs=[
            pl.BlockSpec((TC_BBLK, FSLICE), lambda i: (i, 0)),
            pl.BlockSpec(memory_space=pltpu.MemorySpace.HBM),
        ],
        out_specs=pl.BlockSpec((FSLICE, TC_BBLK), lambda i, s=split: (s, i)),
        out_shape=jax.ShapeDtypeStruct((EMB_DIM, BATCH), jnp.float32),
        input_output_aliases={1: 0},
    )


def kernel(indices, table):
    idx = indices.astype(jnp.int32)
    parts = [_make_sc_gather(k)(idx, table) for k in range(NSPLIT)]
    out_t = _make_tc_transpose(0)(parts[0])
    for k in range(1, NSPLIT):
        out_t = _make_tc_transpose(k)(parts[k], out_t)
    return out_t.T.reshape(BATCH, CHANNEL, IM_H, IM_W)
